# Initial kernel scaffold; baseline (speedup 1.0000x reference)
import jax
import jax.numpy as jnp
from jax.experimental import pallas as pl


def kernel(x, edge_index, batch, Wl1, Wr1, b1, Wl2, Wr2, b2, Wlat, blat, Wdec, bdec, Wd1l, Wd1r, bd1, Wd2l, Wd2r, bd2):
    raise NotImplementedError("write your pallas kernel here")


# speedup vs baseline: 5.4285x; 5.4285x over previous
import functools

import jax
import jax.numpy as jnp
from jax import lax
from jax.experimental import pallas as pl
from jax.experimental.pallas import tpu as pltpu
from jax.experimental.pallas import tpu_sc as plsc

_N = 10000
_E = 320000
_G = 5000
_D = 128
_H = 128
_L = 64

_NC = 2
_NS = 16
_NW = _NC * _NS
_CH = 128
_NCHUNK = _E // _CH
_RPS = 624
_REM = _N - _NS * _RPS
_ZR = _RPS // 6
_SEG = 160
_GP = _NW * _SEG
_LB = 136

_F32 = jnp.float32


def _mesh():
    return plsc.VectorSubcoreMesh(core_axis_name="c", subcore_axis_name="s",
                                  num_cores=_NC, num_subcores=_NS)


def _zero_vec():
    return jnp.zeros((16,), _F32)


@functools.cache
def _agg_kernel(count_only: bool):
    out = jax.ShapeDtypeStruct((_NC, _N, _D), _F32)
    scratch = [
        pltpu.VMEM_SHARED((_N, _D), _F32),
        pltpu.VMEM((_CH,), jnp.int32),
        pltpu.VMEM((_CH,), jnp.int32),
        pltpu.VMEM((_CH, _D), _F32),
        pltpu.SemaphoreType.DMA,
    ]

    def body(x_hbm, src_hbm, dst_hbm, part_hbm, acc, sidx, didx, rows, sem):
        c = lax.axis_index("c")
        s = lax.axis_index("s")
        wid = c * _NS + s

        zf = _zero_vec()

        def zrow_body(j, _):
            for k in range(_D // 16):
                rows[j, pl.ds(k * 16, 16)] = zf
            return 0

        lax.fori_loop(0, _CH, zrow_body, 0)
        for k in range(_RPS // _ZR):
            pltpu.sync_copy(rows.at[pl.ds(0, _ZR)],
                            acc.at[pl.ds(s * _RPS + k * _ZR, _ZR)])

        @pl.when(s == _NS - 1)
        def _():
            pltpu.sync_copy(rows.at[pl.ds(0, _REM)],
                            acc.at[pl.ds(_NS * _RPS, _REM)])

        if count_only:
            one = jnp.ones((16,), _F32)

            def ones_body(j, _):
                for k in range(_D // 16):
                    rows[j, pl.ds(k * 16, 16)] = one
                return 0

            lax.fori_loop(0, _CH, ones_body, 0)

        plsc.subcore_barrier()

        nch = (_NCHUNK - wid + _NW - 1) // _NW

        def step(i, _):
            base = (wid + i * _NW) * _CH
            pltpu.sync_copy(dst_hbm.at[pl.ds(base, _CH)], didx)
            if not count_only:
                pltpu.sync_copy(src_hbm.at[pl.ds(base, _CH)], sidx)
                pltpu.async_copy(x_hbm.at[sidx], rows, sem).wait()
            pltpu.sync_copy(rows, acc.at[didx], add=True)
            return 0

        lax.fori_loop(0, nch, step, 0)
        plsc.subcore_barrier()
        pltpu.sync_copy(acc.at[pl.ds(s * _RPS, _RPS)],
                        part_hbm.at[c, pl.ds(s * _RPS, _RPS)])

        @pl.when(s == _NS - 1)
        def _():
            pltpu.sync_copy(acc.at[pl.ds(_NS * _RPS, _REM)],
                            part_hbm.at[c, pl.ds(_NS * _RPS, _REM)])

    return pl.kernel(body, out_type=out, mesh=_mesh(),
                     scratch_types=scratch)


@functools.cache
def _pool_kernel():
    outs = (jax.ShapeDtypeStruct((_GP, _H), _F32),
            jax.ShapeDtypeStruct((_GP, _H), _F32))
    scratch = [
        pltpu.VMEM((_N + 16,), jnp.int32),
        pltpu.VMEM((_LB, _H), _F32),
        pltpu.VMEM((_SEG, _H), _F32),
        pltpu.VMEM((_SEG, _H), _F32),
        pltpu.VMEM((_SEG, 16), _F32),
    ]

    def body(h_hbm, batch_hbm, pmean_hbm, pmax_hbm,
             batch_v, rows, sums, maxs, cnts):
        c = lax.axis_index("c")
        s = lax.axis_index("s")
        wid = c * _NS + s
        lo = wid * _SEG
        nseg = jnp.minimum(_SEG, _G - lo)

        def batch_at(idx):
            v = batch_v[pl.ds(idx, 16)]
            return v[0]

        pltpu.sync_copy(batch_hbm, batch_v.at[pl.ds(0, _N)])

        zf = _zero_vec()
        neg = jnp.full((16,), -3.4e38, _F32)

        def init_body(j, _):
            for k in range(_H // 16):
                sl = pl.ds(k * 16, 16)
                sums[j, sl] = zf
                maxs[j, sl] = neg
            cnts[j, :] = zf
            return 0

        lax.fori_loop(0, _SEG, init_body, 0)

        def lower_bound(v):
            def bd(_, st):
                lo_, hi_ = st
                live = lo_ < hi_
                mid = (lo_ + hi_) // 2
                pred = batch_at(mid) < v
                return (jnp.where(live & pred, mid + 1, lo_),
                        jnp.where(live & ~pred, mid, hi_))

            return lax.fori_loop(
                0, 14, bd, (jnp.int32(0), jnp.int32(_N)))[0]

        start = lower_bound(lo)
        end = lower_bound(lo + nseg)
        nrows = end - start
        nchk = (nrows + 127) // 128

        one = jnp.ones((16,), _F32)

        def chunk(k, _):
            b0 = start + k * 128
            bl = jnp.minimum((b0 // 8) * 8, _N - _LB)
            delta = b0 - bl
            pltpu.sync_copy(h_hbm.at[pl.ds(bl, _LB)], rows)
            m = jnp.minimum(128, end - b0)

            def row(r, _):
                i = b0 + r
                li = batch_at(i) - lo
                rr = delta + r
                for k2 in range(_H // 16):
                    sl = pl.ds(k2 * 16, 16)
                    v = rows[rr, sl]
                    sums[li, sl] = sums[li, sl] + v
                    maxs[li, sl] = jnp.maximum(maxs[li, sl], v)
                cnts[li, :] = cnts[li, :] + one
                return 0

            lax.fori_loop(0, m, row, 0)
            return 0

        lax.fori_loop(0, nchk, chunk, 0)

        def epilogue(j, _):
            cv = cnts[j, :]
            inv = 1.0 / jnp.maximum(cv, 1.0)
            pos = cv > 0.0
            for k2 in range(_H // 16):
                sl = pl.ds(k2 * 16, 16)
                sums[j, sl] = sums[j, sl] * inv
                maxs[j, sl] = jnp.where(pos, maxs[j, sl], 0.0)
            return 0

        lax.fori_loop(0, _SEG, epilogue, 0)
        pltpu.sync_copy(sums, pmean_hbm.at[pl.ds(lo, _SEG)])
        pltpu.sync_copy(maxs, pmax_hbm.at[pl.ds(lo, _SEG)])

    return pl.kernel(body, out_type=outs, mesh=_mesh(),
                     scratch_types=scratch)


def _dotT(a, w):
    return lax.dot_general(a, w, (((1,), (1,)), ((), ())),
                           preferred_element_type=_F32)


@functools.cache
def _conv_tc(relu: bool):
    blk = 2000

    def body(p_ref, c_ref, x_ref, wl_ref, wr_ref, b_ref, o_ref):
        csum = c_ref[0, :, 0:1] + c_ref[1, :, 0:1]
        inv = 1.0 / jnp.maximum(csum, 1.0)
        mean = (p_ref[0] + p_ref[1]) * inv
        acc = _dotT(mean, wl_ref[...]) + _dotT(x_ref[...], wr_ref[...])
        acc = acc + b_ref[...]
        o_ref[...] = jnp.maximum(acc, 0.0) if relu else acc

    return pl.pallas_call(
        body,
        grid=(_N // blk,),
        in_specs=[
            pl.BlockSpec((2, blk, _D), lambda i: (0, i, 0)),
            pl.BlockSpec((2, blk, _D), lambda i: (0, i, 0)),
            pl.BlockSpec((blk, _D), lambda i: (i, 0)),
            pl.BlockSpec((_H, _D), lambda i: (0, 0)),
            pl.BlockSpec((_H, _D), lambda i: (0, 0)),
            pl.BlockSpec((1, _H), lambda i: (0, 0)),
        ],
        out_specs=pl.BlockSpec((blk, _H), lambda i: (i, 0)),
        out_shape=jax.ShapeDtypeStruct((_N, _H), _F32),
    )


@functools.cache
def _dense_tc():
    blk = 1000

    def body(pm_ref, px_ref, wm_ref, wx_ref, bl_ref, wd_ref, bd_ref,
             emb_ref, z_ref):
        emb = _dotT(pm_ref[...], wm_ref[...]) + _dotT(px_ref[...], wx_ref[...])
        emb = emb + bl_ref[...]
        emb_ref[...] = emb
        z_ref[...] = jnp.maximum(_dotT(emb, wd_ref[...]) + bd_ref[...], 0.0)

    return pl.pallas_call(
        body,
        grid=(_G // blk,),
        in_specs=[
            pl.BlockSpec((blk, _H), lambda i: (i, 0)),
            pl.BlockSpec((blk, _H), lambda i: (i, 0)),
            pl.BlockSpec((_L, _H), lambda i: (0, 0)),
            pl.BlockSpec((_L, _H), lambda i: (0, 0)),
            pl.BlockSpec((1, _L), lambda i: (0, 0)),
            pl.BlockSpec((2 * _H, _L), lambda i: (0, 0)),
            pl.BlockSpec((1, 2 * _H), lambda i: (0, 0)),
        ],
        out_specs=[
            pl.BlockSpec((blk, _L), lambda i: (i, 0)),
            pl.BlockSpec((blk, 2 * _H), lambda i: (i, 0)),
        ],
        out_shape=[
            jax.ShapeDtypeStruct((_G, _L), _F32),
            jax.ShapeDtypeStruct((_G, 2 * _H), _F32),
        ],
    )


def kernel(x, edge_index, batch, Wl1, Wr1, b1, Wl2, Wr2, b2, Wlat, blat,
           Wdec, bdec, Wd1l, Wd1r, bd1, Wd2l, Wd2r, bd2):
    src = edge_index[0]
    dst = edge_index[1]

    cnt = _agg_kernel(True)(x, src, dst)
    part1 = _agg_kernel(False)(x, src, dst)
    h1 = _conv_tc(True)(part1, cnt, x, Wl1, Wr1, b1.reshape(1, _H))
    part2 = _agg_kernel(False)(h1, src, dst)
    h2 = _conv_tc(True)(part2, cnt, h1, Wl2, Wr2, b2.reshape(1, _H))

    pmean_p, pmax_p = _pool_kernel()(h2, batch)
    emb, z = _dense_tc()(pmean_p[:_G], pmax_p[:_G], Wlat[:, :_H],
                         Wlat[:, _H:], blat.reshape(1, _L), Wdec,
                         bdec.reshape(1, 2 * _H))
    z = z.reshape(_N, _H)

    part3 = _agg_kernel(False)(z, src, dst)
    z1 = _conv_tc(True)(part3, cnt, z, Wd1l, Wd1r, bd1.reshape(1, _H))
    part4 = _agg_kernel(False)(z1, src, dst)
    out = _conv_tc(False)(part4, cnt, z1, Wd2l, Wd2r, bd2.reshape(1, _H))
    return (out, emb)


# speedup vs baseline: 7.9670x; 1.4676x over previous
import functools

import jax
import jax.numpy as jnp
from jax import lax
from jax.experimental import pallas as pl
from jax.experimental.pallas import tpu as pltpu
from jax.experimental.pallas import tpu_sc as plsc

_N = 10000
_E = 320000
_G = 5000
_D = 128
_H = 128
_L = 64

_NC = 2
_NS = 16
_NW = _NC * _NS
_CH = 128
_NCHUNK = _E // _CH
_RPS = 624
_REM = _N - _NS * _RPS
_ZR = _RPS // 6
_SEG = 160
_GP = _NW * _SEG
_LB = 136

_F32 = jnp.float32


def _mesh():
    return plsc.VectorSubcoreMesh(core_axis_name="c", subcore_axis_name="s",
                                  num_cores=_NC, num_subcores=_NS)


def _zero_vec():
    return jnp.zeros((16,), _F32)


@functools.cache
def _agg_kernel(count_only: bool):
    out = jax.ShapeDtypeStruct((_NC, _N, _D), _F32)
    scratch = [
        pltpu.VMEM_SHARED((_N, _D), _F32),
        pltpu.VMEM((_CH,), jnp.int32),
        pltpu.VMEM((_CH,), jnp.int32),
        pltpu.VMEM((_CH, _D), _F32),
        pltpu.VMEM((_CH,), jnp.int32),
        pltpu.VMEM((_CH,), jnp.int32),
        pltpu.VMEM((_CH, _D), _F32),
        pltpu.SemaphoreType.DMA,
        pltpu.SemaphoreType.DMA,
    ]
    cpt = _NCHUNK // _NW
    tail = _NCHUNK - cpt * _NW

    def body(x_hbm, src_hbm, dst_hbm, part_hbm, acc,
             sidx0, didx0, rows0, sidx1, didx1, rows1, gsem0, gsem1):
        c = lax.axis_index("c")
        s = lax.axis_index("s")
        wid = c * _NS + s

        zf = _zero_vec()

        def zrow_body(j, _):
            for k in range(_D // 16):
                rows0[j, pl.ds(k * 16, 16)] = zf
            return 0

        lax.fori_loop(0, _CH, zrow_body, 0)
        for k in range(_RPS // _ZR):
            pltpu.sync_copy(rows0.at[pl.ds(0, _ZR)],
                            acc.at[pl.ds(s * _RPS + k * _ZR, _ZR)])

        @pl.when(s == _NS - 1)
        def _():
            pltpu.sync_copy(rows0.at[pl.ds(0, _REM)],
                            acc.at[pl.ds(_NS * _RPS, _REM)])

        if count_only:
            one = jnp.ones((16,), _F32)

            def ones_body(j, _):
                for k in range(_D // 16):
                    rows0[j, pl.ds(k * 16, 16)] = one
                return 0

            lax.fori_loop(0, _CH, ones_body, 0)

        plsc.subcore_barrier()

        if count_only:
            nch = (_NCHUNK - wid + _NW - 1) // _NW

            def step(i, _):
                base = (wid + i * _NW) * _CH
                pltpu.sync_copy(dst_hbm.at[pl.ds(base, _CH)], didx0)
                pltpu.sync_copy(rows0, acc.at[didx0], add=True)
                return 0

            lax.fori_loop(0, nch, step, 0)
        else:
            c0 = wid * cpt

            def load_idx(ci, sidxb, didxb):
                base = ci * _CH
                pltpu.sync_copy(src_hbm.at[pl.ds(base, _CH)], sidxb)
                pltpu.sync_copy(dst_hbm.at[pl.ds(base, _CH)], didxb)

            load_idx(c0, sidx0, didx0)
            pltpu.async_copy(x_hbm.at[sidx0], rows0, gsem0)

            def step2(k, _):
                load_idx(c0 + 2 * k + 1, sidx1, didx1)
                pltpu.async_copy(x_hbm.at[sidx1], rows1, gsem1)
                pltpu.make_async_copy(x_hbm.at[sidx0], rows0, gsem0).wait()
                pltpu.sync_copy(rows0, acc.at[didx0], add=True)

                @pl.when(k < cpt // 2 - 1)
                def _():
                    load_idx(c0 + 2 * k + 2, sidx0, didx0)
                    pltpu.async_copy(x_hbm.at[sidx0], rows0, gsem0)

                pltpu.make_async_copy(x_hbm.at[sidx1], rows1, gsem1).wait()
                pltpu.sync_copy(rows1, acc.at[didx1], add=True)
                return 0

            lax.fori_loop(0, cpt // 2, step2, 0)

            @pl.when(wid < tail)
            def _():
                ci = _NW * cpt + wid
                load_idx(ci, sidx0, didx0)
                pltpu.async_copy(x_hbm.at[sidx0], rows0, gsem0).wait()
                pltpu.sync_copy(rows0, acc.at[didx0], add=True)

        plsc.subcore_barrier()
        pltpu.sync_copy(acc.at[pl.ds(s * _RPS, _RPS)],
                        part_hbm.at[c, pl.ds(s * _RPS, _RPS)])

        @pl.when(s == _NS - 1)
        def _():
            pltpu.sync_copy(acc.at[pl.ds(_NS * _RPS, _REM)],
                            part_hbm.at[c, pl.ds(_NS * _RPS, _REM)])

    return pl.kernel(body, out_type=out, mesh=_mesh(),
                     scratch_types=scratch)


@functools.cache
def _pool_kernel():
    outs = (jax.ShapeDtypeStruct((_GP, _H), _F32),
            jax.ShapeDtypeStruct((_GP, _H), _F32))
    scratch = [
        pltpu.VMEM((_N + 16,), jnp.int32),
        pltpu.VMEM((_LB, _H), _F32),
        pltpu.VMEM((_SEG, _H), _F32),
        pltpu.VMEM((_SEG, _H), _F32),
        pltpu.VMEM((_SEG, 16), _F32),
    ]

    def body(h_hbm, batch_hbm, pmean_hbm, pmax_hbm,
             batch_v, rows, sums, maxs, cnts):
        c = lax.axis_index("c")
        s = lax.axis_index("s")
        wid = c * _NS + s
        lo = wid * _SEG
        nseg = jnp.minimum(_SEG, _G - lo)

        def batch_at(idx):
            v = batch_v[pl.ds(idx, 16)]
            return v[0]

        pltpu.sync_copy(batch_hbm, batch_v.at[pl.ds(0, _N)])

        zf = _zero_vec()
        neg = jnp.full((16,), -3.4e38, _F32)

        def init_body(j, _):
            for k in range(_H // 16):
                sl = pl.ds(k * 16, 16)
                sums[j, sl] = zf
                maxs[j, sl] = neg
            cnts[j, :] = zf
            return 0

        lax.fori_loop(0, _SEG, init_body, 0)

        def lower_bound(v):
            def bd(_, st):
                lo_, hi_ = st
                live = lo_ < hi_
                mid = (lo_ + hi_) // 2
                pred = batch_at(mid) < v
                return (jnp.where(live & pred, mid + 1, lo_),
                        jnp.where(live & ~pred, mid, hi_))

            return lax.fori_loop(
                0, 14, bd, (jnp.int32(0), jnp.int32(_N)))[0]

        start = lower_bound(lo)
        end = lower_bound(lo + nseg)
        nrows = end - start
        nchk = (nrows + 127) // 128

        one = jnp.ones((16,), _F32)

        def chunk(k, _):
            b0 = start + k * 128
            bl = jnp.minimum((b0 // 8) * 8, _N - _LB)
            delta = b0 - bl
            pltpu.sync_copy(h_hbm.at[pl.ds(bl, _LB)], rows)
            m = jnp.minimum(128, end - b0)

            def row(r, _):
                i = b0 + r
                li = batch_at(i) - lo
                rr = delta + r
                for k2 in range(_H // 16):
                    sl = pl.ds(k2 * 16, 16)
                    v = rows[rr, sl]
                    sums[li, sl] = sums[li, sl] + v
                    maxs[li, sl] = jnp.maximum(maxs[li, sl], v)
                cnts[li, :] = cnts[li, :] + one
                return 0

            lax.fori_loop(0, m, row, 0)
            return 0

        lax.fori_loop(0, nchk, chunk, 0)

        def epilogue(j, _):
            cv = cnts[j, :]
            inv = 1.0 / jnp.maximum(cv, 1.0)
            pos = cv > 0.0
            for k2 in range(_H // 16):
                sl = pl.ds(k2 * 16, 16)
                sums[j, sl] = sums[j, sl] * inv
                maxs[j, sl] = jnp.where(pos, maxs[j, sl], 0.0)
            return 0

        lax.fori_loop(0, _SEG, epilogue, 0)
        pltpu.sync_copy(sums, pmean_hbm.at[pl.ds(lo, _SEG)])
        pltpu.sync_copy(maxs, pmax_hbm.at[pl.ds(lo, _SEG)])

    return pl.kernel(body, out_type=outs, mesh=_mesh(),
                     scratch_types=scratch)


def _dotT(a, w):
    return lax.dot_general(a, w, (((1,), (1,)), ((), ())),
                           preferred_element_type=_F32)


@functools.cache
def _conv_tc(relu: bool):
    blk = 2000

    def body(p_ref, c_ref, x_ref, wl_ref, wr_ref, b_ref, o_ref):
        csum = c_ref[0, :, 0:1] + c_ref[1, :, 0:1]
        inv = 1.0 / jnp.maximum(csum, 1.0)
        mean = (p_ref[0] + p_ref[1]) * inv
        acc = _dotT(mean, wl_ref[...]) + _dotT(x_ref[...], wr_ref[...])
        acc = acc + b_ref[...]
        o_ref[...] = jnp.maximum(acc, 0.0) if relu else acc

    return pl.pallas_call(
        body,
        grid=(_N // blk,),
        in_specs=[
            pl.BlockSpec((2, blk, _D), lambda i: (0, i, 0)),
            pl.BlockSpec((2, blk, _D), lambda i: (0, i, 0)),
            pl.BlockSpec((blk, _D), lambda i: (i, 0)),
            pl.BlockSpec((_H, _D), lambda i: (0, 0)),
            pl.BlockSpec((_H, _D), lambda i: (0, 0)),
            pl.BlockSpec((1, _H), lambda i: (0, 0)),
        ],
        out_specs=pl.BlockSpec((blk, _H), lambda i: (i, 0)),
        out_shape=jax.ShapeDtypeStruct((_N, _H), _F32),
    )


@functools.cache
def _dense_tc():
    blk = 1000

    def body(pm_ref, px_ref, wm_ref, wx_ref, bl_ref, wd_ref, bd_ref,
             emb_ref, z_ref):
        emb = _dotT(pm_ref[...], wm_ref[...]) + _dotT(px_ref[...], wx_ref[...])
        emb = emb + bl_ref[...]
        emb_ref[...] = emb
        z_ref[...] = jnp.maximum(_dotT(emb, wd_ref[...]) + bd_ref[...], 0.0)

    return pl.pallas_call(
        body,
        grid=(_G // blk,),
        in_specs=[
            pl.BlockSpec((blk, _H), lambda i: (i, 0)),
            pl.BlockSpec((blk, _H), lambda i: (i, 0)),
            pl.BlockSpec((_L, _H), lambda i: (0, 0)),
            pl.BlockSpec((_L, _H), lambda i: (0, 0)),
            pl.BlockSpec((1, _L), lambda i: (0, 0)),
            pl.BlockSpec((2 * _H, _L), lambda i: (0, 0)),
            pl.BlockSpec((1, 2 * _H), lambda i: (0, 0)),
        ],
        out_specs=[
            pl.BlockSpec((blk, _L), lambda i: (i, 0)),
            pl.BlockSpec((blk, 2 * _H), lambda i: (i, 0)),
        ],
        out_shape=[
            jax.ShapeDtypeStruct((_G, _L), _F32),
            jax.ShapeDtypeStruct((_G, 2 * _H), _F32),
        ],
    )


def kernel(x, edge_index, batch, Wl1, Wr1, b1, Wl2, Wr2, b2, Wlat, blat,
           Wdec, bdec, Wd1l, Wd1r, bd1, Wd2l, Wd2r, bd2):
    src = edge_index[0]
    dst = edge_index[1]

    cnt = _agg_kernel(True)(x, src, dst)
    part1 = _agg_kernel(False)(x, src, dst)
    h1 = _conv_tc(True)(part1, cnt, x, Wl1, Wr1, b1.reshape(1, _H))
    part2 = _agg_kernel(False)(h1, src, dst)
    h2 = _conv_tc(True)(part2, cnt, h1, Wl2, Wr2, b2.reshape(1, _H))

    pmean_p, pmax_p = _pool_kernel()(h2, batch)
    emb, z = _dense_tc()(pmean_p[:_G], pmax_p[:_G], Wlat[:, :_H],
                         Wlat[:, _H:], blat.reshape(1, _L), Wdec,
                         bdec.reshape(1, 2 * _H))
    z = z.reshape(_N, _H)

    part3 = _agg_kernel(False)(z, src, dst)
    z1 = _conv_tc(True)(part3, cnt, z, Wd1l, Wd1r, bd1.reshape(1, _H))
    part4 = _agg_kernel(False)(z1, src, dst)
    out = _conv_tc(False)(part4, cnt, z1, Wd2l, Wd2r, bd2.reshape(1, _H))
    return (out, emb)
